# baseline (speedup 1.0000x reference)
import functools

import jax
import jax.numpy as jnp
from jax import lax
from jax.experimental import pallas as pl
from jax.experimental.pallas import tpu as pltpu
from jax.experimental.pallas import tpu_sc as plsc

_LANES = 16
_CHUNK = 32


def _permute(x, perm):
    dnums = lax.GatherDimensionNumbers(
        offset_dims=(), collapsed_slice_dims=(0,), start_index_map=(0,))
    return lax.gather(x, perm[:, None], dimension_numbers=dnums,
                      slice_sizes=(1,),
                      mode=lax.GatherScatterMode.PROMISE_IN_BOUNDS)


@functools.lru_cache(maxsize=None)
def _build(batch, embed_dim):
    info = plsc.get_sparse_core_info()
    nc, ns = info.num_cores, info.num_subcores
    nw = nc * ns
    b_per_w = batch // nw
    n_chunks = b_per_w // _CHUNK
    n_sub = embed_dim // _LANES

    mesh = plsc.VectorSubcoreMesh(core_axis_name="c", subcore_axis_name="s")

    @functools.partial(
        pl.kernel,
        out_type=jax.ShapeDtypeStruct((batch,), jnp.float32),
        mesh=mesh,
        scratch_types=[
            pltpu.VMEM((b_per_w,), jnp.int32),
            pltpu.VMEM((b_per_w,), jnp.int32),
            pltpu.VMEM((_CHUNK, embed_dim), jnp.float32),
            pltpu.VMEM((_CHUNK, embed_dim), jnp.float32),
            pltpu.VMEM((_CHUNK, embed_dim), jnp.float32),
            pltpu.VMEM((_CHUNK, embed_dim), jnp.float32),
            pltpu.VMEM((embed_dim,), jnp.float32),
            pltpu.VMEM((_LANES,), jnp.float32),
            pltpu.VMEM((b_per_w,), jnp.float32),
            pltpu.SemaphoreType.DMA,
            pltpu.SemaphoreType.DMA,
        ],
    )
    def sc_kernel(uidx_hbm, midx_hbm, utab_hbm, mtab_hbm, w_hbm, b_hbm,
                  out_hbm, uidx_v, midx_v,
                  ubuf0, ubuf1, mbuf0, mbuf1, w_v, b_v, out_v, sem0, sem1):
        wid = lax.axis_index("s") * nc + lax.axis_index("c")
        base = wid * b_per_w

        pltpu.sync_copy(uidx_hbm.at[pl.ds(base, b_per_w)], uidx_v)
        pltpu.sync_copy(midx_hbm.at[pl.ds(base, b_per_w)], midx_v)
        pltpu.sync_copy(w_hbm, w_v)
        pltpu.sync_copy(b_hbm, b_v)

        bias_vec = b_v[...]
        w_vecs = [w_v[pl.ds(j * _LANES, _LANES)] for j in range(n_sub)]
        lane = lax.iota(jnp.int32, _LANES)
        perms = [lane ^ s for s in (8, 4, 2, 1)]

        def fire(c, ubuf, mbuf, sem):
            uvec = uidx_v[pl.ds(c * _CHUNK, _CHUNK)]
            mvec = midx_v[pl.ds(c * _CHUNK, _CHUNK)]
            for k in range(_CHUNK):
                pltpu.async_copy(
                    utab_hbm.at[lax.shift_right_logical(uvec[k], 3),
                                jnp.bitwise_and(uvec[k], 7)],
                    ubuf.at[k], sem)
                pltpu.async_copy(
                    mtab_hbm.at[lax.shift_right_logical(mvec[k], 3),
                                jnp.bitwise_and(mvec[k], 7)],
                    mbuf.at[k], sem)

        def wait(ubuf, mbuf, sem):
            for k in range(_CHUNK):
                pltpu.make_async_copy(
                    utab_hbm.at[0, 0], ubuf.at[k], sem).wait()
                pltpu.make_async_copy(
                    mtab_hbm.at[0, 0], mbuf.at[k], sem).wait()

        def compute(c, ubuf, mbuf):
            for g in range(_CHUNK // _LANES):
                o_vec = jnp.zeros((_LANES,), jnp.float32)
                for kk in range(_LANES):
                    k = g * _LANES + kk
                    acc = None
                    for j in range(n_sub):
                        sl = pl.ds(j * _LANES, _LANES)
                        p = ubuf[k, sl] * mbuf[k, sl] * w_vecs[j]
                        acc = p if acc is None else acc + p
                    for perm in perms:
                        acc = acc + _permute(acc, perm)
                    o_vec = jnp.where(lane == kk, acc, o_vec)
                out_v[pl.ds(c * _CHUNK + g * _LANES, _LANES)] = o_vec + bias_vec

        fire(0, ubuf0, mbuf0, sem0)

        def body(c2, _):
            c = c2 * 2
            fire(c + 1, ubuf1, mbuf1, sem1)
            wait(ubuf0, mbuf0, sem0)
            compute(c, ubuf0, mbuf0)

            @pl.when(c2 < n_chunks // 2 - 1)
            def _():
                fire(c + 2, ubuf0, mbuf0, sem0)

            wait(ubuf1, mbuf1, sem1)
            compute(c + 1, ubuf1, mbuf1)
            return 0

        lax.fori_loop(0, n_chunks // 2, body, 0)

        pltpu.sync_copy(out_v, out_hbm.at[pl.ds(base, b_per_w)])

    return sc_kernel


def kernel(user_idx, movie_idx, user_table, movie_table, fc_w, fc_b):
    batch = user_idx.shape[0]
    embed_dim = user_table.shape[1]
    sc_kernel = _build(batch, embed_dim)

    uidx = user_idx.astype(jnp.int32)
    midx = movie_idx.astype(jnp.int32)
    w = fc_w.reshape(embed_dim)
    b = jnp.broadcast_to(fc_b.reshape(1), (_LANES,))
    ut3 = user_table.reshape(user_table.shape[0] // 8, 8, embed_dim)
    mt3 = movie_table.reshape(movie_table.shape[0] // 8, 8, embed_dim)
    return sc_kernel(uidx, midx, ut3, mt3, w, b)


# speedup vs baseline: 1.0191x; 1.0191x over previous
import functools

import jax
import jax.numpy as jnp
from jax import lax
from jax.experimental import pallas as pl
from jax.experimental.pallas import tpu as pltpu
from jax.experimental.pallas import tpu_sc as plsc

_LANES = 16
_CHUNK = 16


def _permute(x, perm):
    dnums = lax.GatherDimensionNumbers(
        offset_dims=(), collapsed_slice_dims=(0,), start_index_map=(0,))
    return lax.gather(x, perm[:, None], dimension_numbers=dnums,
                      slice_sizes=(1,),
                      mode=lax.GatherScatterMode.PROMISE_IN_BOUNDS)


@functools.lru_cache(maxsize=None)
def _build(batch, embed_dim):
    info = plsc.get_sparse_core_info()
    nc, ns = info.num_cores, info.num_subcores
    nw = nc * ns
    b_per_w = batch // nw
    n_chunks = b_per_w // _CHUNK
    n_sub = embed_dim // _LANES

    mesh = plsc.VectorSubcoreMesh(core_axis_name="c", subcore_axis_name="s")

    @functools.partial(
        pl.kernel,
        out_type=jax.ShapeDtypeStruct((batch,), jnp.float32),
        mesh=mesh,
        scratch_types=[
            pltpu.VMEM((b_per_w,), jnp.int32),
            pltpu.VMEM((b_per_w,), jnp.int32),
            pltpu.VMEM((_CHUNK, embed_dim), jnp.float32),
            pltpu.VMEM((_CHUNK, embed_dim), jnp.float32),
            pltpu.VMEM((_CHUNK, embed_dim), jnp.float32),
            pltpu.VMEM((_CHUNK, embed_dim), jnp.float32),
            pltpu.VMEM((embed_dim,), jnp.float32),
            pltpu.VMEM((_LANES,), jnp.float32),
            pltpu.VMEM((b_per_w,), jnp.float32),
            pltpu.SemaphoreType.DMA,
            pltpu.SemaphoreType.DMA,
        ],
    )
    def sc_kernel(uidx_hbm, midx_hbm, utab_hbm, mtab_hbm, w_hbm, b_hbm,
                  out_hbm, uidx_v, midx_v,
                  ubuf0, ubuf1, mbuf0, mbuf1, w_v, b_v, out_v, sem0, sem1):
        wid = lax.axis_index("s") * nc + lax.axis_index("c")
        base = wid * b_per_w

        pltpu.sync_copy(uidx_hbm.at[pl.ds(base, b_per_w)], uidx_v)
        pltpu.sync_copy(midx_hbm.at[pl.ds(base, b_per_w)], midx_v)
        pltpu.sync_copy(w_hbm, w_v)
        pltpu.sync_copy(b_hbm, b_v)

        bias_vec = b_v[...]
        w_vecs = [w_v[pl.ds(j * _LANES, _LANES)] for j in range(n_sub)]
        lane = lax.iota(jnp.int32, _LANES)
        perms = [lane ^ s for s in (8, 4, 2, 1)]

        def fire(c, ubuf, mbuf, sem):
            uvec = uidx_v[pl.ds(c * _CHUNK, _CHUNK)]
            mvec = midx_v[pl.ds(c * _CHUNK, _CHUNK)]
            for k in range(_CHUNK):
                pltpu.async_copy(
                    utab_hbm.at[lax.shift_right_logical(uvec[k], 3),
                                jnp.bitwise_and(uvec[k], 7)],
                    ubuf.at[k], sem)
                pltpu.async_copy(
                    mtab_hbm.at[lax.shift_right_logical(mvec[k], 3),
                                jnp.bitwise_and(mvec[k], 7)],
                    mbuf.at[k], sem)

        def wait(ubuf, mbuf, sem):
            for k in range(_CHUNK):
                pltpu.make_async_copy(
                    utab_hbm.at[0, 0], ubuf.at[k], sem).wait()
                pltpu.make_async_copy(
                    mtab_hbm.at[0, 0], mbuf.at[k], sem).wait()

        def compute(c, ubuf, mbuf):
            uvec = uidx_v[pl.ds(c * _CHUNK, _CHUNK)]
            mvec = midx_v[pl.ds(c * _CHUNK, _CHUNK)]
            o_vec = jnp.zeros((_LANES,), jnp.float32)
            for k in range(_CHUNK):
                acc = None
                for j in range(n_sub):
                    sl = pl.ds(j * _LANES, _LANES)
                    p = ubuf[k, sl] * mbuf[k, sl] * w_vecs[j]
                    acc = p if acc is None else acc + p
                for perm in perms:
                    acc = acc + _permute(acc, perm)
                o_vec = jnp.where(lane == k, acc, o_vec)
            out_v[pl.ds(c * _CHUNK, _CHUNK)] = o_vec + bias_vec

        fire(0, ubuf0, mbuf0, sem0)

        def body(c2, _):
            c = c2 * 2
            fire(c + 1, ubuf1, mbuf1, sem1)
            wait(ubuf0, mbuf0, sem0)
            compute(c, ubuf0, mbuf0)

            @pl.when(c2 < n_chunks // 2 - 1)
            def _():
                fire(c + 2, ubuf0, mbuf0, sem0)

            wait(ubuf1, mbuf1, sem1)
            compute(c + 1, ubuf1, mbuf1)
            return 0

        lax.fori_loop(0, n_chunks // 2, body, 0)

        pltpu.sync_copy(out_v, out_hbm.at[pl.ds(base, b_per_w)])

    return sc_kernel


def kernel(user_idx, movie_idx, user_table, movie_table, fc_w, fc_b):
    batch = user_idx.shape[0]
    embed_dim = user_table.shape[1]
    sc_kernel = _build(batch, embed_dim)

    uidx = user_idx.astype(jnp.int32)
    midx = movie_idx.astype(jnp.int32)
    w = fc_w.reshape(embed_dim)
    b = jnp.broadcast_to(fc_b.reshape(1), (_LANES,))
    ut3 = user_table.reshape(user_table.shape[0] // 8, 8, embed_dim)
    mt3 = movie_table.reshape(movie_table.shape[0] // 8, 8, embed_dim)
    return sc_kernel(uidx, midx, ut3, mt3, w, b)
